# baseline (device time: 13369 ns/iter reference)
import jax
import jax.numpy as jnp
from jax import lax
from jax.experimental import pallas as pl
from jax.experimental.pallas import tpu as pltpu

B, H, D, BS = 16, 16, 64, 16
NSLOTS = 128
NP = 128
HD = H * D
H2 = H // 2
HD2 = H2 * D
R2 = H2 * B
G = 2
HG = H2 // G
CW = HG * D
RG = HG * B
NC = 2
TC = BS // NC


def kernel(Q, K, V, bt, lens):
    lens1 = lens.reshape(1, B)
    k3 = K.transpose(1, 2, 3, 0).reshape(BS, HD, NP)
    v3 = V.transpose(1, 2, 3, 0).reshape(BS, HD, NP)
    yidx = lax.axis_index("y").astype(jnp.int32).reshape(1)

    def body(y_sref, q_ref, k_ref, v_ref, bt_ref, lens_ref, out_ref,
             s_ref, ck_scr, qb_scr, mparts, lparts, oparts,
             mlsend, osend, mlrecv, orecv, oout,
             sems_sml, sems_rml, sems_so, sems_ro, sem_out):
        i = pl.program_id(0)
        my_x = lax.axis_index("x")
        my_y = lax.axis_index("y")
        peers = [(1 - my_x, my_y), (my_x, 1 - my_y), (1 - my_x, 1 - my_y)]
        barrier = pltpu.get_barrier_semaphore()

        @pl.when(i == 0)
        def _():
            for nbr in peers:
                pl.semaphore_signal(barrier, inc=1, device_id=nbr,
                                    device_id_type=pl.DeviceIdType.MESH)
            slot = lax.broadcasted_iota(jnp.int32, (B, NSLOTS, NP), 1)
            page = lax.broadcasted_iota(jnp.int32, (B, NSLOTS, NP), 2)
            btl = bt_ref[...] - my_x * NP
            lens_col = jnp.swapaxes(lens_ref[...], 0, 1)
            hit = ((btl[:, :, None] == page)
                   & (slot < lens_col[:, :, None]))
            ck_scr[...] = jnp.sum(hit.astype(jnp.float32), axis=1)
            for g in range(G):
                q_g = jnp.concatenate(
                    [q_ref[:, 0, g * HG + hl, :] for hl in range(HG)],
                    axis=1)
                qrep = jnp.concatenate([q_g] * HG, axis=0)
                rowh = lax.broadcasted_iota(jnp.int32, (RG, CW), 0) // B
                colh = lax.broadcasted_iota(jnp.int32, (RG, CW), 1) // D
                qb_scr[g, :, :] = jnp.where(
                    rowh == colh, qrep, 0.0).astype(jnp.bfloat16)

        kb = k_ref[...].astype(jnp.bfloat16)
        vb = v_ref[...].astype(jnp.bfloat16)

        for t in range(TC):
            for g in range(G):
                s_ref[t, g * RG:(g + 1) * RG, :] = lax.dot_general(
                    qb_scr[g], kb[t, g * CW:(g + 1) * CW, :],
                    (((1,), (0,)), ((), ())),
                    preferred_element_type=jnp.float32) * (D ** -0.5)

        s4 = s_ref[...].reshape(TC, H2, B, NP)
        m_i = jnp.max(jnp.max(s4, axis=3), axis=0)
        p4 = (jnp.exp(s4 - m_i[None, :, :, None]).astype(jnp.bfloat16)
              * ck_scr[...][None, None, :, :].astype(jnp.bfloat16))
        l_i = jnp.sum(jnp.sum(p4.astype(jnp.float32), axis=3), axis=0)
        mparts[i, :, :] = m_i
        lparts[i, :, :] = l_i

        pb = p4.reshape(TC, R2, NP)
        for g in range(G):
            o_g = lax.dot_general(
                pb[0, g * RG:(g + 1) * RG, :],
                vb[0, g * CW:(g + 1) * CW, :],
                (((1,), (1,)), ((), ())),
                preferred_element_type=jnp.float32)
            for t in range(1, TC):
                o_g = o_g + lax.dot_general(
                    pb[t, g * RG:(g + 1) * RG, :],
                    vb[t, g * CW:(g + 1) * CW, :],
                    (((1,), (1,)), ((), ())),
                    preferred_element_type=jnp.float32)
            for hl in range(HG):
                oparts[i, g * HG + hl, :, :] = (
                    o_g[hl * B:(hl + 1) * B, hl * D:(hl + 1) * D])

        @pl.when(i == NC - 1)
        def _():
            m_all = mparts[...]
            m_loc = jnp.max(m_all, axis=0)
            a = jnp.exp(m_all - m_loc[None, :, :])
            l_loc = jnp.sum(lparts[...] * a, axis=0)
            o_loc = jnp.sum(oparts[...] * a[:, :, :, None], axis=0)
            mlsend[0, :, :] = m_loc
            mlsend[1, :, :] = l_loc
            osend[...] = o_loc

            pl.semaphore_wait(barrier, 3)
            rdmas = []
            for j, nbr in enumerate(peers):
                rdmas.append(pltpu.make_async_remote_copy(
                    src_ref=mlsend, dst_ref=mlrecv.at[j],
                    send_sem=sems_sml.at[j], recv_sem=sems_rml.at[j],
                    device_id=nbr, device_id_type=pl.DeviceIdType.MESH))
                rdmas.append(pltpu.make_async_remote_copy(
                    src_ref=osend, dst_ref=orecv.at[j],
                    send_sem=sems_so.at[j], recv_sem=sems_ro.at[j],
                    device_id=nbr, device_id_type=pl.DeviceIdType.MESH))
            for r in rdmas:
                r.start()
            for r in rdmas:
                r.wait()

            def merge(m0, l0, o0, m1, l1, o1):
                mm = jnp.maximum(m0, m1)
                a0 = jnp.exp(m0 - mm)
                a1 = jnp.exp(m1 - mm)
                ll = l0 * a0 + l1 * a1
                return (o0 * a0[:, :, None] + o1 * a1[:, :, None]) \
                    / ll[:, :, None]

            oo_mine = merge(m_loc, l_loc, o_loc,
                            mlrecv[0, 0], mlrecv[0, 1], orecv[0])
            oo_other = merge(mlrecv[1, 0], mlrecv[1, 1], orecv[1],
                             mlrecv[2, 0], mlrecv[2, 1], orecv[2])

            lo = jnp.concatenate([oo_mine, oo_other], axis=0)
            hi = jnp.concatenate([oo_other, oo_mine], axis=0)
            full = jnp.where(my_y == 0, lo, hi)
            oout[...] = jnp.swapaxes(full, 0, 1).reshape(B, 1, H, D)
            cp = pltpu.make_async_copy(oout, out_ref, sem_out)
            cp.start()
            cp.wait()

    grid_spec = pltpu.PrefetchScalarGridSpec(
        num_scalar_prefetch=1,
        grid=(NC,),
        in_specs=[
            pl.BlockSpec((B, 1, H2, D), lambda i, y: (0, 0, y[0], 0)),
            pl.BlockSpec((TC, HD2, NP), lambda i, y: (i, y[0], 0)),
            pl.BlockSpec((TC, HD2, NP), lambda i, y: (i, y[0], 0)),
            pl.BlockSpec((B, NSLOTS), lambda i, y: (0, 0)),
            pl.BlockSpec((1, B), lambda i, y: (0, 0)),
        ],
        out_specs=pl.BlockSpec(memory_space=pltpu.MemorySpace.HBM),
        scratch_shapes=[
            pltpu.VMEM((TC, R2, NP), jnp.float32),
            pltpu.VMEM((B, NP), jnp.float32),
            pltpu.VMEM((G, RG, CW), jnp.bfloat16),
            pltpu.VMEM((NC, H2, B), jnp.float32),
            pltpu.VMEM((NC, H2, B), jnp.float32),
            pltpu.VMEM((NC, H2, B, D), jnp.float32),
            pltpu.VMEM((2, H2, B), jnp.float32),
            pltpu.VMEM((H2, B, D), jnp.float32),
            pltpu.VMEM((3, 2, H2, B), jnp.float32),
            pltpu.VMEM((3, H2, B, D), jnp.float32),
            pltpu.VMEM((B, 1, H, D), jnp.float32),
            pltpu.SemaphoreType.DMA((3,)),
            pltpu.SemaphoreType.DMA((3,)),
            pltpu.SemaphoreType.DMA((3,)),
            pltpu.SemaphoreType.DMA((3,)),
            pltpu.SemaphoreType.DMA,
        ],
    )

    return pl.pallas_call(
        body,
        grid_spec=grid_spec,
        out_shape=jax.ShapeDtypeStruct((B, 1, H, D), jnp.float32),
        compiler_params=pltpu.CompilerParams(
            collective_id=0, vmem_limit_bytes=100 * 1024 * 1024),
    )(yidx, Q, k3, v3, bt, lens1)
